# baseline (speedup 1.0000x reference)
import functools
import math

import jax
import jax.numpy as jnp
from jax.experimental import pallas as pl

_EMBED_DIM = 64
_LOG10000 = math.log(10000.0)


def _body(var_ref, out_ref, *, chunk):
    E = _EMBED_DIM
    t0 = pl.program_id(0) * chunk
    pos = (t0 + jax.lax.broadcasted_iota(jnp.int32, (chunk, E), 0)).astype(
        jnp.float32)
    e_idx = jax.lax.broadcasted_iota(jnp.int32, (chunk, E), 1)
    k = (e_idx >> 1).astype(jnp.float32)
    freq = jnp.exp(k * (-2.0 * _LOG10000 / E))
    angle = pos * freq
    pe = jnp.where(e_idx & 1 == 0, jnp.sin(angle), jnp.cos(angle))
    out_ref[0] = var_ref[...][None, :, :] + pe[:, None, :]


def kernel(x, var_table):
    B, T, D = x.shape
    E = _EMBED_DIM
    chunk = 40
    s = pl.pallas_call(
        functools.partial(_body, chunk=chunk),
        grid=(T // chunk,),
        in_specs=[pl.BlockSpec((D, E), lambda i: (0, 0))],
        out_specs=pl.BlockSpec((1, chunk, D, E), lambda i: (0, i, 0, 0)),
        out_shape=jax.ShapeDtypeStruct((1, T, D, E), jnp.float32),
    )(var_table)
    return jnp.broadcast_to(s, (B, T, D, E))
